# baseline (speedup 1.0000x reference)
import functools

import jax
import jax.numpy as jnp
from jax import lax
from jax.experimental import pallas as pl
from jax.experimental.pallas import tpu as pltpu
from jax.experimental.pallas import tpu_sc as plsc

K = 16
C = 896
RK = 128
RS = 256
RM = 128
PAD_B = 1 << 20
IBIG = 1 << 30
BIGV = 3.0e38

_PC = pl.pallas_call
_HIGH = lax.Precision.DEFAULT


def _dot(a, b):
    return lax.dot_general(a, b, (((1,), (0,)), ((), ())),
                           precision=_HIGH,
                           preferred_element_type=jnp.float32)


def _mm_body(x_ref, w_ref, o1_ref, o2_ref):
    d = o2_ref.shape[-1]
    y = _dot(x_ref[...], w_ref[...])
    o1_ref[...] = y[:, :2 * d]
    o2_ref[...] = y[:, 2 * d:]


def _stage1(xp, wct):
    np_, d = xp.shape
    return _PC(
        _mm_body,
        grid=(np_ // RS,),
        in_specs=[
            pl.BlockSpec((RS, d), lambda i: (i, 0)),
            pl.BlockSpec(wct.shape, lambda i: (0, 0)),
        ],
        out_specs=[
            pl.BlockSpec((RS, 2 * d), lambda i: (i, 0)),
            pl.BlockSpec((RS, d), lambda i: (i, 0)),
        ],
        out_shape=[
            jax.ShapeDtypeStruct((np_, 2 * d), jnp.float32),
            jax.ShapeDtypeStruct((np_, d), jnp.float32),
        ],
    )(xp, wct)


def _knn_body(c, i0, lo_ref, nt_ref, posr_ref, prow_ref, batr_ref, brow_ref,
              nbrT_ref, vbuf, ibuf):
    i = i0 + pl.program_id(0)
    np_ = posr_ref.shape[0]
    lo = lo_ref[pl.program_id(0)]
    nt = nt_ref[pl.program_id(0)]
    vbuf[0:K, :] = jnp.full((K, RK), jnp.inf, jnp.float32)
    ibuf[...] = jnp.full((K, RK), IBIG, jnp.int32)
    brow = brow_ref[...]
    row_gid = i * RK + lax.broadcasted_iota(jnp.int32, (1, RK), 1)

    def tile_body(j, _):
        start = pl.multiple_of(jnp.minimum(lo + j * c, np_ - c), 8)
        bc = batr_ref[pl.ds(start, c), 0:1]
        d2 = jnp.zeros((c, RK), jnp.float32)
        for d in range(3):
            diff = posr_ref[pl.ds(start, c), d:d + 1] - prow_ref[d:d + 1, :]
            d2 = d2 + diff * diff
        cand_gid = start + lax.broadcasted_iota(jnp.int32, (c, 1), 0)
        bad = (bc != brow) | (cand_gid == row_gid)
        vt = jnp.where(bad, BIGV, d2)
        vb = vbuf[0:K, :]
        ib = ibuf[...]
        ms, sels = [], []
        for _t in range(K):
            m = jnp.minimum(jnp.min(vt, axis=0, keepdims=True),
                            jnp.min(vb, axis=0, keepdims=True))
            selt = jnp.min(jnp.where(vt == m, cand_gid, IBIG), axis=0,
                           keepdims=True)
            selb = jnp.min(jnp.where(vb == m, ib, IBIG), axis=0,
                           keepdims=True)
            sel = jnp.minimum(selb, selt)
            vt = jnp.where((vt == m) & (cand_gid == sel), jnp.inf, vt)
            vb = jnp.where((vb == m) & (ib == sel), jnp.inf, vb)
            ms.append(m)
            sels.append(sel)
        vbuf[0:K, :] = jnp.concatenate(ms, axis=0)
        ibuf[...] = jnp.concatenate(sels, axis=0)
        return 0

    lax.fori_loop(0, nt, tile_body, 0)
    nbrT_ref[...] = ibuf[...]


def _knn(posr, posc, batr, batc, lo_a, nt, i0, nbh):
    np_ = posr.shape[0]
    c = min(C, np_)
    return _PC(
        functools.partial(_knn_body, c, i0),
        grid=(nbh,),
        in_specs=[
            pl.BlockSpec(memory_space=pltpu.SMEM),
            pl.BlockSpec(memory_space=pltpu.SMEM),
            pl.BlockSpec((np_, 4), lambda i: (0, 0)),
            pl.BlockSpec((4, RK), lambda i: (0, i0 + i)),
            pl.BlockSpec((np_, 1), lambda i: (0, 0)),
            pl.BlockSpec((1, RK), lambda i: (0, i0 + i)),
        ],
        out_specs=pl.BlockSpec((K, RK), lambda i: (0, i)),
        out_shape=jax.ShapeDtypeStruct((K, nbh * RK), jnp.int32),
        scratch_shapes=[
            pltpu.VMEM((K, RK), jnp.float32),
            pltpu.VMEM((K, RK), jnp.int32),
        ],
    )(lo_a, nt, posr, posc, batr, batc)


def _gather(b_arr, idx_flat):
    ne = idx_flat.shape[0]
    d = b_arr.shape[1]
    nw = 32
    epw = ne // nw
    ch = min(128, epw)
    mesh = plsc.VectorSubcoreMesh(core_axis_name="c", subcore_axis_name="s")

    nch = epw // ch

    @functools.partial(
        pl.kernel,
        mesh=mesh,
        out_type=jax.ShapeDtypeStruct((ne, d), jnp.float32),
        scratch_types=[
            pltpu.VMEM((ch,), jnp.int32),
            pltpu.VMEM((ch,), jnp.int32),
            pltpu.VMEM((ch, d), jnp.float32),
            pltpu.VMEM((ch, d), jnp.float32),
            pltpu.SemaphoreType.DMA,
            pltpu.SemaphoreType.DMA,
        ],
    )
    def k(b_hbm, idx_hbm, out_hbm, i0, i1, r0, r1, s0, s1):
        wid = lax.axis_index("s") * 2 + lax.axis_index("c")
        base = wid * epw

        if nch % 2 != 0:
            @pl.loop(0, nch)
            def _(c):
                off = base + c * ch
                pltpu.sync_copy(idx_hbm.at[pl.ds(off, ch)], i0)
                pltpu.async_copy(b_hbm.at[i0], r0, s0).wait()
                pltpu.sync_copy(r0, out_hbm.at[pl.ds(off, ch)])
        else:
            pltpu.sync_copy(idx_hbm.at[pl.ds(base, ch)], i0)
            pltpu.async_copy(b_hbm.at[i0], r0, s0)

            @pl.loop(0, nch, step=2)
            def _(c):
                off = base + c * ch
                pltpu.sync_copy(idx_hbm.at[pl.ds(off + ch, ch)], i1)
                pltpu.async_copy(b_hbm.at[i1], r1, s1)
                pltpu.make_async_copy(b_hbm.at[i0], r0, s0).wait()
                pltpu.sync_copy(r0, out_hbm.at[pl.ds(off, ch)])

                @pl.when(c + 2 < nch)
                def _():
                    pltpu.sync_copy(idx_hbm.at[pl.ds(off + 2 * ch, ch)], i0)
                    pltpu.async_copy(b_hbm.at[i0], r0, s0)

                pltpu.make_async_copy(b_hbm.at[i1], r1, s1).wait()
                pltpu.sync_copy(r1, out_hbm.at[pl.ds(off + ch, ch)])

    return k(b_arr, idx_flat)


def _mlp_body(y_ref, e_ref, w2t_ref, b1_ref, b2_ref, bn_ref, o_ref):
    d = o_ref.shape[-1]
    nk = e_ref.shape[0]
    a = y_ref[:, :d] + b1_ref[...]
    acc = jnp.full((RM, d), -jnp.inf, jnp.float32)
    for k in range(nk):
        h = jnp.maximum(a + e_ref[k], 0.0)
        acc = jnp.maximum(acc, _dot(h, w2t_ref[...]))
    o_ref[...] = acc + b2_ref[...] + y_ref[:, d:] + bn_ref[...]


def _mlp(y1h, e_km, w2t, b1, b2, bn):
    nph, d2w = y1h.shape
    d = d2w // 2
    nk = e_km.shape[0]
    full = lambda *s: pl.BlockSpec(s, lambda i: (0,) * len(s))
    return _PC(
        _mlp_body,
        grid=(nph // RM,),
        in_specs=[
            pl.BlockSpec((RM, 2 * d), lambda i: (i, 0)),
            pl.BlockSpec((nk, RM, d), lambda i: (0, i, 0)),
            full(d, d), full(1, d), full(1, d), full(1, d),
        ],
        out_specs=pl.BlockSpec((RM, d), lambda i: (i, 0)),
        out_shape=jax.ShapeDtypeStruct((nph, d), jnp.float32),
    )(y1h, e_km, w2t, b1, b2, bn)


def kernel(x, pos, batch, W1, b1, W2, b2, Wn, bn):
    n, d = x.shape
    np_ = -(-n // RS) * RS
    pad = np_ - n

    xp = jnp.pad(x, ((0, pad), (0, 0)))
    posr = jnp.pad(pos, ((0, pad), (0, 1)))
    posc = posr.T
    batp = jnp.pad(batch.astype(jnp.int32), (0, pad), constant_values=PAD_B)
    batr = batp[:, None]
    batc = batp[None, :]

    iota = jnp.arange(np_, dtype=jnp.int32)
    first = jnp.concatenate([jnp.ones((1,), bool), batp[1:] != batp[:-1]])
    last = jnp.concatenate([batp[1:] != batp[:-1], jnp.ones((1,), bool)])
    ss = lax.cummax(jnp.where(first, iota, 0))
    le = lax.cummin(jnp.where(last, iota, np_)[::-1])[::-1]
    se = le + 1
    sizes = (se - ss).reshape(np_ // RK, RK)
    widen = jnp.min(sizes, axis=1) < (K + 1)
    lo = ss.reshape(np_ // RK, RK)[:, 0]
    hi = se.reshape(np_ // RK, RK)[:, -1]
    c_eff = min(C, np_)
    lo_a = jnp.where(widen, 0, (lo // 8) * 8).astype(jnp.int32)
    hi_w = jnp.where(widen, np_, hi)
    nt = ((hi_w - lo_a + c_eff - 1) // c_eff).astype(jnp.int32)

    w1a = W1[:, :d]
    w1b = W1[:, d:]
    wct = jnp.concatenate([(w1a - w1b).T, Wn.T, w1b.T], axis=1)

    y1, b_arr = _stage1(xp, wct)

    nhalves = 2
    nbh = np_ // RK // nhalves
    nph = nbh * RK
    w2t = W2.T
    b1r, b2r, bnr = b1[None, :], b2[None, :], bn[None, :]
    outs = []
    for h in range(nhalves):
        sl = slice(h * nbh, (h + 1) * nbh)
        nbrT_h = _knn(posr, posc, batr, batc, lo_a[sl], nt[sl],
                      h * nbh, nbh)
        e_h = _gather(b_arr, nbrT_h.reshape(-1)).reshape(K, nph, d)
        outs.append(_mlp(y1[h * nph:(h + 1) * nph], e_h, w2t, b1r, b2r, bnr))
    out = jnp.concatenate(outs, axis=0)
    return out[:n]


# speedup vs baseline: 1.0027x; 1.0027x over previous
import functools

import jax
import jax.numpy as jnp
from jax import lax
from jax.experimental import pallas as pl
from jax.experimental.pallas import tpu as pltpu
from jax.experimental.pallas import tpu_sc as plsc

K = 16
C = 896
RK = 128
RS = 256
RM = 128
PAD_B = 1 << 20
IBIG = 1 << 30
BIGV = 3.0e38

_PC = pl.pallas_call
_HIGH = lax.Precision.DEFAULT


def _dot(a, b):
    return lax.dot_general(a, b, (((1,), (0,)), ((), ())),
                           precision=_HIGH,
                           preferred_element_type=jnp.float32)


def _mm_body(x_ref, w_ref, o1_ref, o2_ref):
    d = o2_ref.shape[-1]
    y = _dot(x_ref[...], w_ref[...])
    o1_ref[...] = y[:, :2 * d]
    o2_ref[...] = y[:, 2 * d:]


def _stage1(xp, wct):
    np_, d = xp.shape
    return _PC(
        _mm_body,
        grid=(np_ // RS,),
        in_specs=[
            pl.BlockSpec((RS, d), lambda i: (i, 0)),
            pl.BlockSpec(wct.shape, lambda i: (0, 0)),
        ],
        out_specs=[
            pl.BlockSpec((RS, 2 * d), lambda i: (i, 0)),
            pl.BlockSpec((RS, d), lambda i: (i, 0)),
        ],
        out_shape=[
            jax.ShapeDtypeStruct((np_, 2 * d), jnp.float32),
            jax.ShapeDtypeStruct((np_, d), jnp.float32),
        ],
    )(xp, wct)


def _knn_body(c, i0, lo_ref, nt_ref, posr_ref, prow_ref, batr_ref, brow_ref,
              nbrT_ref, vbuf, ibuf):
    i = i0 + pl.program_id(0)
    np_ = posr_ref.shape[0]
    lo = lo_ref[pl.program_id(0)]
    nt = nt_ref[pl.program_id(0)]
    vbuf[0:K, :] = jnp.full((K, RK), jnp.inf, jnp.float32)
    ibuf[...] = jnp.full((K, RK), IBIG, jnp.int32)
    brow = brow_ref[...]
    row_gid = i * RK + lax.broadcasted_iota(jnp.int32, (1, RK), 1)

    def tile_body(j, _):
        start = pl.multiple_of(jnp.minimum(lo + j * c, np_ - c), 8)
        bc = batr_ref[pl.ds(start, c), 0:1]
        d2 = jnp.zeros((c, RK), jnp.float32)
        for d in range(3):
            diff = posr_ref[pl.ds(start, c), d:d + 1] - prow_ref[d:d + 1, :]
            d2 = d2 + diff * diff
        cand_gid = start + lax.broadcasted_iota(jnp.int32, (c, 1), 0)
        bad = (bc != brow) | (cand_gid == row_gid)
        vt = jnp.where(bad, BIGV, d2)
        vb = vbuf[0:K, :]
        ib = ibuf[...]
        ms, sels = [], []
        for _t in range(K):
            m = jnp.minimum(jnp.min(vt, axis=0, keepdims=True),
                            jnp.min(vb, axis=0, keepdims=True))
            selt = jnp.min(jnp.where(vt == m, cand_gid, IBIG), axis=0,
                           keepdims=True)
            selb = jnp.min(jnp.where(vb == m, ib, IBIG), axis=0,
                           keepdims=True)
            sel = jnp.minimum(selb, selt)
            vt = jnp.where((vt == m) & (cand_gid == sel), jnp.inf, vt)
            vb = jnp.where((vb == m) & (ib == sel), jnp.inf, vb)
            ms.append(m)
            sels.append(sel)
        vbuf[0:K, :] = jnp.concatenate(ms, axis=0)
        ibuf[...] = jnp.concatenate(sels, axis=0)
        return 0

    lax.fori_loop(0, nt, tile_body, 0)
    nbrT_ref[...] = ibuf[...]


def _knn(posr, posc, batr, batc, lo_a, nt, i0, nbh):
    np_ = posr.shape[0]
    c = min(C, np_)
    return _PC(
        functools.partial(_knn_body, c, i0),
        grid=(nbh,),
        in_specs=[
            pl.BlockSpec(memory_space=pltpu.SMEM),
            pl.BlockSpec(memory_space=pltpu.SMEM),
            pl.BlockSpec((np_, 4), lambda i: (0, 0)),
            pl.BlockSpec((4, RK), lambda i: (0, i0 + i)),
            pl.BlockSpec((np_, 1), lambda i: (0, 0)),
            pl.BlockSpec((1, RK), lambda i: (0, i0 + i)),
        ],
        out_specs=pl.BlockSpec((K, RK), lambda i: (0, i)),
        out_shape=jax.ShapeDtypeStruct((K, nbh * RK), jnp.int32),
        scratch_shapes=[
            pltpu.VMEM((K, RK), jnp.float32),
            pltpu.VMEM((K, RK), jnp.int32),
        ],
    )(lo_a, nt, posr, posc, batr, batc)


def _gather(b_arr, idx_flat):
    ne = idx_flat.shape[0]
    d = b_arr.shape[1]
    nw = 32
    epw = ne // nw
    ch = min(256, epw)
    mesh = plsc.VectorSubcoreMesh(core_axis_name="c", subcore_axis_name="s")

    @functools.partial(
        pl.kernel,
        mesh=mesh,
        out_type=jax.ShapeDtypeStruct((ne, d), jnp.float32),
        scratch_types=[
            pltpu.VMEM((ch,), jnp.int32),
            pltpu.VMEM((ch, d), jnp.float32),
            pltpu.SemaphoreType.DMA,
        ],
    )
    def k(b_hbm, idx_hbm, out_hbm, idx_v, rows_v, sem):
        wid = lax.axis_index("s") * 2 + lax.axis_index("c")
        base = wid * epw

        @pl.loop(0, epw // ch)
        def _(c):
            off = base + c * ch
            pltpu.sync_copy(idx_hbm.at[pl.ds(off, ch)], idx_v)
            pltpu.async_copy(b_hbm.at[idx_v], rows_v, sem).wait()
            pltpu.sync_copy(rows_v, out_hbm.at[pl.ds(off, ch)])

    return k(b_arr, idx_flat)


def _mlp_body(y_ref, e_ref, w2t_ref, b1_ref, b2_ref, bn_ref, o_ref):
    d = o_ref.shape[-1]
    nk = e_ref.shape[0]
    a = y_ref[:, :d] + b1_ref[...]
    acc = jnp.full((RM, d), -jnp.inf, jnp.float32)
    for k in range(nk):
        h = jnp.maximum(a + e_ref[k], 0.0)
        acc = jnp.maximum(acc, _dot(h, w2t_ref[...]))
    o_ref[...] = acc + b2_ref[...] + y_ref[:, d:] + bn_ref[...]


def _mlp(y1h, e_km, w2t, b1, b2, bn):
    nph, d2w = y1h.shape
    d = d2w // 2
    nk = e_km.shape[0]
    full = lambda *s: pl.BlockSpec(s, lambda i: (0,) * len(s))
    return _PC(
        _mlp_body,
        grid=(nph // RM,),
        in_specs=[
            pl.BlockSpec((RM, 2 * d), lambda i: (i, 0)),
            pl.BlockSpec((nk, RM, d), lambda i: (0, i, 0)),
            full(d, d), full(1, d), full(1, d), full(1, d),
        ],
        out_specs=pl.BlockSpec((RM, d), lambda i: (i, 0)),
        out_shape=jax.ShapeDtypeStruct((nph, d), jnp.float32),
    )(y1h, e_km, w2t, b1, b2, bn)


def kernel(x, pos, batch, W1, b1, W2, b2, Wn, bn):
    n, d = x.shape
    np_ = -(-n // RS) * RS
    pad = np_ - n

    xp = jnp.pad(x, ((0, pad), (0, 0)))
    posr = jnp.pad(pos, ((0, pad), (0, 1)))
    posc = posr.T
    batp = jnp.pad(batch.astype(jnp.int32), (0, pad), constant_values=PAD_B)
    batr = batp[:, None]
    batc = batp[None, :]

    iota = jnp.arange(np_, dtype=jnp.int32)
    first = jnp.concatenate([jnp.ones((1,), bool), batp[1:] != batp[:-1]])
    last = jnp.concatenate([batp[1:] != batp[:-1], jnp.ones((1,), bool)])
    ss = lax.cummax(jnp.where(first, iota, 0))
    le = lax.cummin(jnp.where(last, iota, np_)[::-1])[::-1]
    se = le + 1
    sizes = (se - ss).reshape(np_ // RK, RK)
    widen = jnp.min(sizes, axis=1) < (K + 1)
    lo = ss.reshape(np_ // RK, RK)[:, 0]
    hi = se.reshape(np_ // RK, RK)[:, -1]
    c_eff = min(C, np_)
    lo_a = jnp.where(widen, 0, (lo // 8) * 8).astype(jnp.int32)
    hi_w = jnp.where(widen, np_, hi)
    nt = ((hi_w - lo_a + c_eff - 1) // c_eff).astype(jnp.int32)

    w1a = W1[:, :d]
    w1b = W1[:, d:]
    wct = jnp.concatenate([(w1a - w1b).T, Wn.T, w1b.T], axis=1)

    y1, b_arr = _stage1(xp, wct)

    nhalves = 2
    nbh = np_ // RK // nhalves
    nph = nbh * RK
    w2t = W2.T
    b1r, b2r, bnr = b1[None, :], b2[None, :], bn[None, :]
    outs = []
    for h in range(nhalves):
        sl = slice(h * nbh, (h + 1) * nbh)
        nbrT_h = _knn(posr, posc, batr, batc, lo_a[sl], nt[sl],
                      h * nbh, nbh)
        e_h = _gather(b_arr, nbrT_h.reshape(-1)).reshape(K, nph, d)
        outs.append(_mlp(y1[h * nph:(h + 1) * nph], e_h, w2t, b1r, b2r, bnr))
    out = jnp.concatenate(outs, axis=0)
    return out[:n]
